# baseline (speedup 1.0000x reference)
import functools

import jax
import jax.numpy as jnp
from jax import lax
from jax.experimental import pallas as pl
from jax.experimental.pallas import tpu as pltpu
from jax.experimental.pallas import tpu_sc as plsc

NUM_FEAT = 1000000
NF = 26
H = 64
BATCH = 4096
NC, NS, L = 2, 16, 16
NW = NC * NS
BPW = BATCH // NW
MW = H // 4
NV = H // L


def _emb_body(xT_hbm, w_hbm, m_hbm, cb_hbm, out_hbm,
              idx_v, cb_v, wrows, mrows, obuf, gsem):
    wid = lax.axis_index("s") * NC + lax.axis_index("c")
    b0 = wid * BPW
    pltpu.sync_copy(xT_hbm.at[:, pl.ds(b0, BPW)], idx_v)
    pltpu.sync_copy(cb_hbm, cb_v)

    lanes = lax.iota(jnp.int32, L)
    shamt = (lanes & 3) * 8
    widx = [lanes // 4 + NV * k for k in range(NV)]

    def field_body(f, carry):
        cw = pltpu.async_copy(w_hbm.at[idx_v.at[f]], wrows, gsem)
        cm = pltpu.async_copy(m_hbm.at[idx_v.at[f]], mrows, gsem)
        cw.wait()
        cm.wait()
        cbv = [cb_v[f, pl.ds(L * k, L)] for k in range(NV)]

        def row_body(i, c2):
            ri = jnp.full((L,), i, jnp.int32)
            for k in range(NV):
                mwords = plsc.load_gather(mrows, [ri, widx[k]])
                mf = ((mwords >> shamt) & 1).astype(jnp.float32)
                wv = wrows[i, pl.ds(L * k, L)]
                obuf[i, pl.ds(L * k, L)] = wv + mf * (cbv[k] - wv)
            return c2

        lax.fori_loop(0, BPW, row_body, 0)
        pltpu.sync_copy(obuf, out_hbm.at[pl.ds(b0, BPW), f])
        return carry

    lax.fori_loop(0, NF, field_body, 0)


@jax.jit
def _emb_call(xT, weight, mask32, codebook):
    mesh = plsc.VectorSubcoreMesh(
        core_axis_name="c", subcore_axis_name="s")
    f = functools.partial(
        pl.kernel,
        out_type=jax.ShapeDtypeStruct((BATCH, NF, H), jnp.float32),
        mesh=mesh,
        scratch_types=[
            pltpu.VMEM((NF, BPW), jnp.int32),
            pltpu.VMEM((NF, H), jnp.float32),
            pltpu.VMEM((BPW, H), jnp.float32),
            pltpu.VMEM((BPW, MW), jnp.int32),
            pltpu.VMEM((BPW, H), jnp.float32),
            pltpu.SemaphoreType.DMA,
        ],
        compiler_params=pltpu.CompilerParams(
            use_tc_tiling_on_sc=False, needs_layout_passes=False),
    )(_emb_body)
    return f(xT, weight, mask32, codebook)


def kernel(x, weight, codebook_mask, codebook):
    xT = x.T
    m8 = codebook_mask.view(jnp.uint8)
    m32 = lax.bitcast_convert_type(
        m8.reshape(NUM_FEAT, MW, 4), jnp.int32)
    return _emb_call(xT, weight, m32, codebook)
